# baseline (speedup 1.0000x reference)
import functools

import jax
import jax.numpy as jnp
from jax import lax
from jax.experimental import pallas as pl
from jax.experimental.pallas import tpu as pltpu
from jax.experimental.pallas import tpu_sc as plsc


def _proj_body(emb_ref, w_ref, b_ref, out_ref):
    out_ref[...] = lax.dot_general(
        emb_ref[...], w_ref[...],
        dimension_numbers=(((1,), (1,)), ((), ())),
        preferred_element_type=jnp.float32,
    ) + b_ref[...]


def _project(emb, w, b_vec):
    v, o = emb.shape[0], w.shape[0]
    return pl.pallas_call(
        _proj_body,
        out_shape=jax.ShapeDtypeStruct((v, o), jnp.float32),
    )(emb, w, b_vec.reshape(1, o))


@functools.lru_cache(maxsize=None)
def _make_gather(D, B, V):
    info = plsc.get_sparse_core_info()
    nc, ns = info.num_cores, info.num_subcores
    nw = nc * ns
    b_per_w = B // nw
    ch = 64
    nbuf = 3
    n_chunks = b_per_w // ch
    assert n_chunks * ch == b_per_w and ch % 8 == 0
    assert (n_chunks - 4) % nbuf == 0
    mesh = plsc.VectorSubcoreMesh(core_axis_name="c", subcore_axis_name="s")

    @functools.partial(
        pl.kernel,
        mesh=mesh,
        out_type=jax.ShapeDtypeStruct((B, D), jnp.float32),
        scratch_types=[
            pltpu.VMEM((b_per_w,), jnp.int32)]
            + [pltpu.VMEM((ch, D), jnp.float32)] * nbuf
            + [pltpu.SemaphoreType.DMA] * (2 * nbuf),
    )
    def gather(table_hbm, idx_hbm, out_hbm, idx_v, *bufs_sems):
        bufs = bufs_sems[:nbuf]
        gsems = bufs_sems[nbuf:2 * nbuf]
        wsems = bufs_sems[2 * nbuf:]
        wid = lax.axis_index("s") * nc + lax.axis_index("c")
        base = wid * b_per_w

        def gstart(i, p):
            pltpu.make_async_copy(
                table_hbm.at[idx_v.at[pl.ds(i * ch, ch)]], bufs[p],
                gsems[p]).start()

        def gwait(p):
            pltpu.make_async_copy(
                table_hbm.at[idx_v.at[pl.ds(0, ch)]], bufs[p], gsems[p]).wait()

        def wstart(i, p):
            off = pl.multiple_of(base + i * ch, 8)
            pltpu.make_async_copy(
                bufs[p], out_hbm.at[pl.ds(off, ch)], wsems[p]).start()

        def wwait(p):
            pltpu.make_async_copy(
                bufs[p], out_hbm.at[pl.ds(base, ch)], wsems[p]).wait()

        def step(i, p, first=False, last=False):
            gwait(p)
            wstart(i, p)
            if not last:
                q = (p + 2) % nbuf
                if not first:
                    wwait(q)
                gstart(i + 2, q)

        pltpu.sync_copy(idx_hbm.at[pl.ds(base, b_per_w)], idx_v)

        gstart(0, 0)
        gstart(1, 1)
        step(0, 0, first=True)
        step(1, 1)

        def body(j, carry):
            i = nbuf * j + 2
            step(i, 2)
            step(i + 1, 0)
            step(i + 2, 1)
            return carry

        lax.fori_loop(0, (n_chunks - 4) // nbuf, body, 0)

        step(n_chunks - 2, (n_chunks - 2) % nbuf, last=True)
        step(n_chunks - 1, (n_chunks - 1) % nbuf, last=True)
        for k in (3, 2, 1):
            wwait((n_chunks - k) % nbuf)

    return gather


def kernel(input_ids, emb_table, W, b):
    bsz, seq = input_ids.shape
    o = W.shape[0]
    projected = _project(emb_table, W, b)
    idx = input_ids.reshape(-1).astype(jnp.int32)
    out = _make_gather(o, bsz * seq, emb_table.shape[0])(projected, idx)
    return out.reshape(bsz, seq, o)


# speedup vs baseline: 1.0011x; 1.0011x over previous
import functools

import jax
import jax.numpy as jnp
from jax import lax
from jax.experimental import pallas as pl
from jax.experimental.pallas import tpu as pltpu
from jax.experimental.pallas import tpu_sc as plsc


def _proj_body(emb_ref, w_ref, b_ref, out_ref):
    out_ref[...] = lax.dot_general(
        emb_ref[...], w_ref[...],
        dimension_numbers=(((1,), (1,)), ((), ())),
        preferred_element_type=jnp.float32,
    ) + b_ref[...]


def _project(emb, w, b_vec):
    v, o = emb.shape[0], w.shape[0]
    return pl.pallas_call(
        _proj_body,
        out_shape=jax.ShapeDtypeStruct((v, o), jnp.float32),
    )(emb, w, b_vec.reshape(1, o))


@functools.lru_cache(maxsize=None)
def _make_gather(D, B):
    info = plsc.get_sparse_core_info()
    nc, ns = info.num_cores, info.num_subcores
    nw = nc * ns
    b_per_w = B // nw
    ch = 64
    nbuf = 3
    n_chunks = b_per_w // ch
    assert n_chunks * ch == b_per_w and ch % 8 == 0
    assert (n_chunks - 4) % nbuf == 0
    mesh = plsc.VectorSubcoreMesh(core_axis_name="c", subcore_axis_name="s")

    @functools.partial(
        pl.kernel,
        mesh=mesh,
        out_type=jax.ShapeDtypeStruct((B, D), jnp.float32),
        scratch_types=[
            pltpu.VMEM((b_per_w,), jnp.int32)]
            + [pltpu.VMEM((ch, D), jnp.float32)] * nbuf
            + [pltpu.SemaphoreType.DMA] * (2 * nbuf),
    )
    def gather(table_hbm, idx_hbm, out_hbm, idx_v, *bufs_sems):
        bufs = bufs_sems[:nbuf]
        gsems = bufs_sems[nbuf:2 * nbuf]
        wsems = bufs_sems[2 * nbuf:]
        wid = lax.axis_index("s") * nc + lax.axis_index("c")
        base = wid * b_per_w

        def gstart(i, p):
            pltpu.make_async_copy(
                table_hbm.at[idx_v.at[pl.ds(i * ch, ch)]], bufs[p],
                gsems[p]).start()

        def gwait(p):
            pltpu.make_async_copy(
                table_hbm.at[idx_v.at[pl.ds(0, ch)]], bufs[p], gsems[p]).wait()

        def wstart(i, p):
            off = pl.multiple_of(base + i * ch, 8)
            pltpu.make_async_copy(
                bufs[p], out_hbm.at[pl.ds(off, ch)], wsems[p]).start()

        def wwait(p):
            pltpu.make_async_copy(
                bufs[p], out_hbm.at[pl.ds(base, ch)], wsems[p]).wait()

        def step(i, p, first=False, last=False):
            gwait(p)
            wstart(i, p)
            if not last:
                q = (p + 2) % nbuf
                if not first:
                    wwait(q)
                gstart(i + 2, q)

        pltpu.sync_copy(idx_hbm.at[pl.ds(base, b_per_w)], idx_v)

        gstart(0, 0)
        gstart(1, 1)
        step(0, 0, first=True)
        step(1, 1)

        def body(j, carry):
            i = nbuf * j + 2
            step(i, 2)
            step(i + 1, 0)
            step(i + 2, 1)
            return carry

        lax.fori_loop(0, (n_chunks - 4) // nbuf, body, 0)

        step(n_chunks - 2, (n_chunks - 2) % nbuf, last=True)
        step(n_chunks - 1, (n_chunks - 1) % nbuf, last=True)
        for k in (3, 2, 1):
            wwait((n_chunks - k) % nbuf)

    return gather


def kernel(input_ids, emb_table, W, b):
    bsz, seq = input_ids.shape
    o = W.shape[0]
    projected = _project(emb_table, W, b)
    idx = input_ids.reshape(-1).astype(jnp.int32)
    out = _make_gather(o, bsz * seq)(projected, idx)
    return out.reshape(bsz, seq, o)
